# Initial kernel scaffold; baseline (speedup 1.0000x reference)
import functools

import jax
import jax.numpy as jnp
from jax import lax
from jax.experimental import pallas as pl
from jax.experimental.pallas import tpu as pltpu
from jax.experimental.pallas import tpu_sc as plsc

N = 10000
E = 320000
D_IN = 128
D_HID = 128
D_OUT = 41
D_OUT_PAD = 48

NC = 2
NS = 16
NW = NC * NS
EPW = E // NW

_mesh = plsc.VectorSubcoreMesh(core_axis_name="c", subcore_axis_name="s")


@functools.partial(
    pl.kernel,
    mesh=_mesh,
    out_type=[
        jax.ShapeDtypeStruct((NW, N), jnp.float32),
        jax.ShapeDtypeStruct((NW, N), jnp.float32),
    ],
    scratch_types=[
        pltpu.VMEM((EPW,), jnp.int32),
        pltpu.VMEM((EPW,), jnp.int32),
        pltpu.VMEM((N,), jnp.float32),
        pltpu.VMEM((N,), jnp.float32),
    ],
)
def _degree_kernel(src_hbm, dst_hbm, dsrc_out, ddst_out, src_v, dst_v, hs_v, hd_v):
    c = lax.axis_index("c")
    s = lax.axis_index("s")
    wid = s * NC + c
    base = wid * EPW
    pltpu.sync_copy(src_hbm.at[pl.ds(base, EPW)], src_v)
    pltpu.sync_copy(dst_hbm.at[pl.ds(base, EPW)], dst_v)

    zero16 = jnp.zeros((16,), jnp.float32)

    def zbody(i, carry):
        hs_v[pl.ds(i * 16, 16)] = zero16
        hd_v[pl.ds(i * 16, 16)] = zero16
        return carry

    lax.fori_loop(0, N // 16, zbody, 0)

    ones16 = jnp.ones((16,), jnp.float32)

    def body(i, carry):
        sv = src_v[pl.ds(i * 16, 16)]
        dv = dst_v[pl.ds(i * 16, 16)]
        plsc.addupdate_scatter(hs_v, [sv], ones16)
        plsc.addupdate_scatter(hd_v, [dv], ones16)
        return carry

    lax.fori_loop(0, EPW // 16, body, 0)

    pltpu.sync_copy(hs_v, dsrc_out.at[wid])
    pltpu.sync_copy(hd_v, ddst_out.at[wid])


def _make_agg(d, chunk):
    nch = EPW // chunk
    rows_per_tile = N // NS
    zrows = rows_per_tile // 5

    @functools.partial(
        pl.kernel,
        mesh=_mesh,
        out_type=jax.ShapeDtypeStruct((NC, N, d), jnp.float32),
        scratch_types=[
            pltpu.VMEM((chunk,), jnp.int32),
            pltpu.VMEM((chunk,), jnp.int32),
            pltpu.VMEM((chunk, d), jnp.float32),
            pltpu.VMEM((zrows, d), jnp.float32),
            pltpu.VMEM_SHARED((N, d), jnp.float32),
            pltpu.SemaphoreType.DMA,
        ],
    )
    def _agg_kernel(h_hbm, src_hbm, dst_hbm, out_hbm, src_v, dst_v, rows_v, zbuf, acc_sh, sem):
        c = lax.axis_index("c")
        s = lax.axis_index("s")
        wid = s * NC + c

        zero16 = jnp.zeros((16,), jnp.float32)
        dl = d // 16

        def zfill(i, carry):
            r = i // dl
            col = lax.rem(i, dl)
            zbuf[r, pl.ds(col * 16, 16)] = zero16
            return carry

        lax.fori_loop(0, zrows * dl, zfill, 0)

        row0 = s * rows_per_tile

        def zcopy(j, carry):
            pltpu.sync_copy(zbuf, acc_sh.at[pl.ds(row0 + j * zrows, zrows)])
            return carry

        lax.fori_loop(0, rows_per_tile // zrows, zcopy, 0)
        plsc.subcore_barrier()

        base = wid * EPW

        def body(i, carry):
            pltpu.sync_copy(src_hbm.at[pl.ds(base + i * chunk, chunk)], src_v)
            pltpu.sync_copy(dst_hbm.at[pl.ds(base + i * chunk, chunk)], dst_v)
            pltpu.async_copy(h_hbm.at[src_v], rows_v, sem).wait()
            pltpu.sync_copy(rows_v, acc_sh.at[dst_v], add=True)
            return carry

        lax.fori_loop(0, nch, body, 0)
        plsc.subcore_barrier()

        def wcopy(j, carry):
            r0 = row0 + j * zrows
            pltpu.sync_copy(acc_sh.at[pl.ds(r0, zrows)], zbuf)
            pltpu.sync_copy(zbuf, out_hbm.at[c, pl.ds(r0, zrows)])
            return carry

        lax.fori_loop(0, rows_per_tile // zrows, wcopy, 0)

    return _agg_kernel


_agg128 = _make_agg(D_HID, 80)
_agg48 = _make_agg(D_OUT_PAD, 80)


def _norm(deg):
    return jnp.where(deg > 0, lax.rsqrt(jnp.maximum(deg, 1.0)), 0.0)


_RB = 1000
_GRID = N // _RB


def _tc1_body(x_ref, dp_ref, w1_ref, h1_ref):
    deg = jnp.sum(dp_ref[...], axis=0)
    no = _norm(deg)
    h1_ref[...] = jnp.dot(
        x_ref[...] * no[:, None], w1_ref[...], preferred_element_type=jnp.float32
    )


def _tc1_call(x, dsrc_p, W1):
    return pl.pallas_call(
        _tc1_body,
        grid=(_GRID,),
        in_specs=[
            pl.BlockSpec((_RB, D_IN), lambda i: (i, 0)),
            pl.BlockSpec((NW, _RB), lambda i: (0, i)),
            pl.BlockSpec((D_IN, D_HID), lambda i: (0, 0)),
        ],
        out_specs=pl.BlockSpec((_RB, D_HID), lambda i: (i, 0)),
        out_shape=jax.ShapeDtypeStruct((N, D_HID), jnp.float32),
    )(x, dsrc_p, W1)


def _tc2_body(agg_ref, dps_ref, dpd_ref, b1_ref, w2_ref, h2_ref):
    agg = agg_ref[0] + agg_ref[1]
    ni = _norm(jnp.sum(dpd_ref[...], axis=0))
    no = _norm(jnp.sum(dps_ref[...], axis=0))
    o1 = jnp.maximum(agg * ni[:, None] + b1_ref[0][None, :], 0.0)
    h2_ref[...] = jnp.dot(
        o1 * no[:, None], w2_ref[...], preferred_element_type=jnp.float32
    )


def _tc2_call(agg1, dsrc_p, ddst_p, b1t, W2p):
    return pl.pallas_call(
        _tc2_body,
        grid=(_GRID,),
        in_specs=[
            pl.BlockSpec((NC, _RB, D_HID), lambda i: (0, i, 0)),
            pl.BlockSpec((NW, _RB), lambda i: (0, i)),
            pl.BlockSpec((NW, _RB), lambda i: (0, i)),
            pl.BlockSpec((8, D_HID), lambda i: (0, 0)),
            pl.BlockSpec((D_HID, D_OUT_PAD), lambda i: (0, 0)),
        ],
        out_specs=pl.BlockSpec((_RB, D_OUT_PAD), lambda i: (i, 0)),
        out_shape=jax.ShapeDtypeStruct((N, D_OUT_PAD), jnp.float32),
    )(agg1, dsrc_p, ddst_p, b1t, W2p)


def _tc3_body(agg_ref, dpd_ref, b2_ref, out_ref):
    agg = agg_ref[0] + agg_ref[1]
    ni = _norm(jnp.sum(dpd_ref[...], axis=0))
    out_ref[...] = agg * ni[:, None] + b2_ref[0][None, :]


def _tc3_call(agg2, ddst_p, b2t):
    return pl.pallas_call(
        _tc3_body,
        grid=(_GRID,),
        in_specs=[
            pl.BlockSpec((NC, _RB, D_OUT_PAD), lambda i: (0, i, 0)),
            pl.BlockSpec((NW, _RB), lambda i: (0, i)),
            pl.BlockSpec((8, D_OUT_PAD), lambda i: (0, 0)),
        ],
        out_specs=pl.BlockSpec((_RB, D_OUT_PAD), lambda i: (i, 0)),
        out_shape=jax.ShapeDtypeStruct((N, D_OUT_PAD), jnp.float32),
    )(agg2, ddst_p, b2t)


@jax.jit
def kernel(x, edge_index, W1, b1, W2, b2):
    src = edge_index[0].astype(jnp.int32)
    dst = edge_index[1].astype(jnp.int32)

    dsrc_p, ddst_p = _degree_kernel(src, dst)

    h1 = _tc1_call(x, dsrc_p, W1)
    agg1 = _agg128(h1, src, dst)

    W2p = jnp.pad(W2, ((0, 0), (0, D_OUT_PAD - D_OUT)))
    b1t = jnp.tile(b1[None, :], (8, 1))
    h2 = _tc2_call(agg1, dsrc_p, ddst_p, b1t, W2p)
    agg2 = _agg48(h2, src, dst)

    b2t = jnp.tile(jnp.pad(b2, (0, D_OUT_PAD - D_OUT))[None, :], (8, 1))
    outp = _tc3_call(agg2, ddst_p, b2t)
    return outp[:, :D_OUT]


# speedup vs baseline: 6.0021x; 6.0021x over previous
import functools

import jax
import jax.numpy as jnp
from jax import lax
from jax.experimental import pallas as pl
from jax.experimental.pallas import tpu as pltpu
from jax.experimental.pallas import tpu_sc as plsc

N = 10000
E = 320000
D_IN = 128
D_HID = 128
D_OUT = 41
D_OUT_PAD = 48

NC = 2
NS = 16
NW = NC * NS
EPW = E // NW

_mesh = plsc.VectorSubcoreMesh(core_axis_name="c", subcore_axis_name="s")


@functools.partial(
    pl.kernel,
    mesh=_mesh,
    compiler_params=pltpu.CompilerParams(needs_layout_passes=False),
    out_type=[
        jax.ShapeDtypeStruct((NW, N), jnp.float32),
        jax.ShapeDtypeStruct((NW, N), jnp.float32),
    ],
    scratch_types=[
        pltpu.VMEM((EPW,), jnp.int32),
        pltpu.VMEM((EPW,), jnp.int32),
        pltpu.VMEM((N,), jnp.float32),
        pltpu.VMEM((N,), jnp.float32),
    ],
)
def _degree_kernel(src_hbm, dst_hbm, dsrc_out, ddst_out, src_v, dst_v, hs_v, hd_v):
    c = lax.axis_index("c")
    s = lax.axis_index("s")
    wid = s * NC + c
    base = wid * EPW
    pltpu.sync_copy(src_hbm.at[pl.ds(base, EPW)], src_v)
    pltpu.sync_copy(dst_hbm.at[pl.ds(base, EPW)], dst_v)

    zero16 = jnp.zeros((16,), jnp.float32)

    def zbody(i, carry):
        hs_v[pl.ds(i * 16, 16)] = zero16
        hd_v[pl.ds(i * 16, 16)] = zero16
        return carry

    lax.fori_loop(0, N // 16, zbody, 0)

    ones16 = jnp.ones((16,), jnp.float32)

    def body(i, carry):
        sv = src_v[pl.ds(i * 16, 16)]
        dv = dst_v[pl.ds(i * 16, 16)]
        plsc.addupdate_scatter(hs_v, [sv], ones16)
        plsc.addupdate_scatter(hd_v, [dv], ones16)
        return carry

    lax.fori_loop(0, EPW // 16, body, 0)

    pltpu.sync_copy(hs_v, dsrc_out.at[wid])
    pltpu.sync_copy(hd_v, ddst_out.at[wid])


def _make_agg(d, chunk):
    nch = EPW // chunk
    nrowch = N // 16
    kmax = -(-nrowch // NS)

    @functools.partial(
        pl.kernel,
        mesh=_mesh,
        compiler_params=pltpu.CompilerParams(
            needs_layout_passes=False, use_tc_tiling_on_sc=False
        ),
        out_type=jax.ShapeDtypeStruct((NC, N, d), jnp.float32),
        scratch_types=[
            pltpu.VMEM((chunk,), jnp.int32),
            pltpu.VMEM((chunk,), jnp.int32),
            pltpu.VMEM((chunk, d), jnp.float32),
            pltpu.VMEM((16, d), jnp.float32),
            pltpu.VMEM_SHARED((N, d), jnp.float32),
            pltpu.SemaphoreType.DMA,
        ],
    )
    def _agg_kernel(h_hbm, src_hbm, dst_hbm, out_hbm, src_v, dst_v, rows_v, zbuf, acc_sh, sem):
        c = lax.axis_index("c")
        s = lax.axis_index("s")
        wid = s * NC + c

        zero16 = jnp.zeros((16,), jnp.float32)
        dl = d // 16

        def zfill(i, carry):
            r = i // dl
            col = lax.rem(i, dl)
            zbuf[r, pl.ds(col * 16, 16)] = zero16
            return carry

        lax.fori_loop(0, 16 * dl, zfill, 0)

        def zcopy(k, carry):
            ch = k * NS + s

            @pl.when(ch < nrowch)
            def _():
                pltpu.sync_copy(zbuf, acc_sh.at[pl.ds(ch * 16, 16)])

            return carry

        lax.fori_loop(0, kmax, zcopy, 0)
        plsc.subcore_barrier()

        base = wid * EPW

        def body(i, carry):
            pltpu.sync_copy(src_hbm.at[pl.ds(base + i * chunk, chunk)], src_v)
            pltpu.sync_copy(dst_hbm.at[pl.ds(base + i * chunk, chunk)], dst_v)
            pltpu.async_copy(h_hbm.at[src_v], rows_v, sem).wait()
            pltpu.sync_copy(rows_v, acc_sh.at[dst_v], add=True)
            return carry

        lax.fori_loop(0, nch, body, 0)
        plsc.subcore_barrier()

        def wcopy(k, carry):
            ch = k * NS + s

            @pl.when(ch < nrowch)
            def _():
                r0 = ch * 16
                pltpu.sync_copy(acc_sh.at[pl.ds(r0, 16)], zbuf)
                pltpu.sync_copy(zbuf, out_hbm.at[c, pl.ds(r0, 16)])

            return carry

        lax.fori_loop(0, kmax, wcopy, 0)

    return _agg_kernel


_agg128 = _make_agg(D_HID, 80)
_agg48 = _make_agg(D_OUT_PAD, 80)


def _norm(deg):
    return jnp.where(deg > 0, lax.rsqrt(jnp.maximum(deg, 1.0)), 0.0)


_RB = 1000
_GRID = N // _RB


def _tc1_body(x_ref, dp_ref, w1_ref, h1_ref):
    no = _norm(jnp.sum(dp_ref[0], axis=0))
    h1_ref[...] = jnp.dot(
        x_ref[...] * no[:, None], w1_ref[...], preferred_element_type=jnp.float32
    )


def _tc1_call(x, dsrc_p, W1):
    return pl.pallas_call(
        _tc1_body,
        grid=(_GRID,),
        in_specs=[
            pl.BlockSpec((_RB, D_IN), lambda i: (i, 0)),
            pl.BlockSpec((1, NW, _RB), lambda i: (i, 0, 0)),
            pl.BlockSpec((D_IN, D_HID), lambda i: (0, 0)),
        ],
        out_specs=pl.BlockSpec((_RB, D_HID), lambda i: (i, 0)),
        out_shape=jax.ShapeDtypeStruct((N, D_HID), jnp.float32),
    )(x, dsrc_p, W1)


def _tc2_body(agg_ref, dps_ref, dpd_ref, b1_ref, w2_ref, h2_ref):
    agg = agg_ref[0] + agg_ref[1]
    ni = _norm(jnp.sum(dpd_ref[0], axis=0))
    no = _norm(jnp.sum(dps_ref[0], axis=0))
    o1 = jnp.maximum(agg * ni[:, None] + b1_ref[0][None, :], 0.0)
    h2_ref[...] = jnp.dot(
        o1 * no[:, None], w2_ref[...], preferred_element_type=jnp.float32
    )


def _tc2_call(agg1, dsrc_p, ddst_p, b1t, W2p):
    return pl.pallas_call(
        _tc2_body,
        grid=(_GRID,),
        in_specs=[
            pl.BlockSpec((NC, _RB, D_HID), lambda i: (0, i, 0)),
            pl.BlockSpec((1, NW, _RB), lambda i: (i, 0, 0)),
            pl.BlockSpec((1, NW, _RB), lambda i: (i, 0, 0)),
            pl.BlockSpec((8, D_HID), lambda i: (0, 0)),
            pl.BlockSpec((D_HID, D_OUT_PAD), lambda i: (0, 0)),
        ],
        out_specs=pl.BlockSpec((_RB, D_OUT_PAD), lambda i: (i, 0)),
        out_shape=jax.ShapeDtypeStruct((N, D_OUT_PAD), jnp.float32),
    )(agg1, dsrc_p, ddst_p, b1t, W2p)


def _tc3_body(agg_ref, dpd_ref, b2_ref, out_ref):
    agg = agg_ref[0] + agg_ref[1]
    ni = _norm(jnp.sum(dpd_ref[0], axis=0))
    out_ref[...] = agg * ni[:, None] + b2_ref[0][None, :]


def _tc3_call(agg2, ddst_p, b2t):
    return pl.pallas_call(
        _tc3_body,
        grid=(_GRID,),
        in_specs=[
            pl.BlockSpec((NC, _RB, D_OUT_PAD), lambda i: (0, i, 0)),
            pl.BlockSpec((1, NW, _RB), lambda i: (i, 0, 0)),
            pl.BlockSpec((8, D_OUT_PAD), lambda i: (0, 0)),
        ],
        out_specs=pl.BlockSpec((_RB, D_OUT_PAD), lambda i: (i, 0)),
        out_shape=jax.ShapeDtypeStruct((N, D_OUT_PAD), jnp.float32),
    )(agg2, ddst_p, b2t)


@jax.jit
def kernel(x, edge_index, W1, b1, W2, b2):
    src = edge_index[0].astype(jnp.int32)
    dst = edge_index[1].astype(jnp.int32)

    dsrc_p, ddst_p = _degree_kernel(src, dst)
    dsrc_p = dsrc_p.reshape(NW, _GRID, _RB).transpose(1, 0, 2)
    ddst_p = ddst_p.reshape(NW, _GRID, _RB).transpose(1, 0, 2)

    h1 = _tc1_call(x, dsrc_p, W1)
    agg1 = _agg128(h1, src, dst)

    W2p = jnp.pad(W2, ((0, 0), (0, D_OUT_PAD - D_OUT)))
    b1t = jnp.tile(b1[None, :], (8, 1))
    h2 = _tc2_call(agg1, dsrc_p, ddst_p, b1t, W2p)
    agg2 = _agg48(h2, src, dst)

    b2t = jnp.tile(jnp.pad(b2, (0, D_OUT_PAD - D_OUT))[None, :], (8, 1))
    outp = _tc3_call(agg2, ddst_p, b2t)
    return outp[:, :D_OUT]


# speedup vs baseline: 9.9592x; 1.6593x over previous
import functools

import jax
import jax.numpy as jnp
from jax import lax
from jax.experimental import pallas as pl
from jax.experimental.pallas import tpu as pltpu
from jax.experimental.pallas import tpu_sc as plsc

N = 10000
E = 320000
D_IN = 128
D_HID = 128
D_OUT = 41
D_OUT_PAD = 48

NC = 2
NS = 16
NW = NC * NS
EPW = E // NW

_mesh = plsc.VectorSubcoreMesh(core_axis_name="c", subcore_axis_name="s")


@functools.partial(
    pl.kernel,
    mesh=_mesh,
    compiler_params=pltpu.CompilerParams(needs_layout_passes=False),
    out_type=[
        jax.ShapeDtypeStruct((NW, N), jnp.float32),
        jax.ShapeDtypeStruct((NW, N), jnp.float32),
    ],
    scratch_types=[
        pltpu.VMEM((EPW,), jnp.int32),
        pltpu.VMEM((EPW,), jnp.int32),
        pltpu.VMEM((N,), jnp.float32),
        pltpu.VMEM((N,), jnp.float32),
    ],
)
def _degree_kernel(src_hbm, dst_hbm, dsrc_out, ddst_out, src_v, dst_v, hs_v, hd_v):
    c = lax.axis_index("c")
    s = lax.axis_index("s")
    wid = s * NC + c
    base = wid * EPW
    pltpu.sync_copy(src_hbm.at[pl.ds(base, EPW)], src_v)
    pltpu.sync_copy(dst_hbm.at[pl.ds(base, EPW)], dst_v)

    zero16 = jnp.zeros((16,), jnp.float32)

    def zbody(i, carry):
        hs_v[pl.ds(i * 16, 16)] = zero16
        hd_v[pl.ds(i * 16, 16)] = zero16
        return carry

    lax.fori_loop(0, N // 16, zbody, 0)

    ones16 = jnp.ones((16,), jnp.float32)

    def body(i, carry):
        sv = src_v[pl.ds(i * 16, 16)]
        dv = dst_v[pl.ds(i * 16, 16)]
        plsc.addupdate_scatter(hs_v, [sv], ones16)
        plsc.addupdate_scatter(hd_v, [dv], ones16)
        return carry

    lax.fori_loop(0, EPW // 16, body, 0)

    pltpu.sync_copy(hs_v, dsrc_out.at[wid])
    pltpu.sync_copy(hd_v, ddst_out.at[wid])


def _make_agg(d, chunk):
    nch = EPW // chunk
    ngrp = -(-nch // 3)
    nrowch = N // 16
    kmax = -(-nrowch // NS)

    @functools.partial(
        pl.kernel,
        mesh=_mesh,
        compiler_params=pltpu.CompilerParams(
            needs_layout_passes=False, use_tc_tiling_on_sc=False
        ),
        out_type=jax.ShapeDtypeStruct((NC, N, d), jnp.float32),
        scratch_types=[
            pltpu.VMEM((3, chunk), jnp.int32),
            pltpu.VMEM((3, chunk), jnp.int32),
            [pltpu.VMEM((chunk, d), jnp.float32) for _ in range(3)],
            pltpu.VMEM((16, d), jnp.float32),
            pltpu.VMEM_SHARED((N, d), jnp.float32),
            [pltpu.SemaphoreType.DMA for _ in range(3)],
            [pltpu.SemaphoreType.DMA for _ in range(3)],
        ],
    )
    def _agg_kernel(h_hbm, src_hbm, dst_hbm, out_hbm, src_c, dst_c, rows, zbuf, acc_sh, gsem, ssem):
        c = lax.axis_index("c")
        s = lax.axis_index("s")
        wid = s * NC + c

        zero16 = jnp.zeros((16,), jnp.float32)
        dl = d // 16

        def zfill(i, carry):
            r = i // dl
            col = lax.rem(i, dl)
            zbuf[r, pl.ds(col * 16, 16)] = zero16
            return carry

        lax.fori_loop(0, 16 * dl, zfill, 0)

        def zcopy(k, carry):
            ch = k * NS + s

            @pl.when(ch < nrowch)
            def _():
                pltpu.sync_copy(zbuf, acc_sh.at[pl.ds(ch * 16, 16)])

            return carry

        lax.fori_loop(0, kmax, zcopy, 0)
        plsc.subcore_barrier()

        def _idx_load(j, b):
            pltpu.sync_copy(src_hbm.at[wid, j], src_c.at[b])
            pltpu.sync_copy(dst_hbm.at[wid, j], dst_c.at[b])

        def _gather_start(j, b):
            del j
            pltpu.async_copy(h_hbm.at[src_c.at[b]], rows[b], gsem[b])

        def _gather_wait(j, b):
            del j
            pltpu.make_async_copy(h_hbm.at[src_c.at[b]], rows[b], gsem[b]).wait()

        def _scatter_start(j, b):
            del j
            pltpu.async_copy(rows[b], acc_sh.at[dst_c.at[b]], ssem[b], add=True)

        def _scatter_wait(j, b):
            del j
            pltpu.make_async_copy(rows[b], acc_sh.at[dst_c.at[b]], ssem[b]).wait()

        _idx_load(0, 0)
        _gather_start(0, 0)
        _idx_load(1, 1)
        _gather_start(1, 1)

        def group(g, carry):
            for b3 in range(3):
                j = g * 3 + b3
                nb = (b3 + 2) % 3

                @pl.when(jnp.logical_and(j >= 1, j + 2 < nch))
                def _():
                    _scatter_wait(j - 1, nb)
                    _idx_load(j + 2, nb)
                    _gather_start(j + 2, nb)

                @pl.when(jnp.logical_and(j < 1, j + 2 < nch))
                def _():
                    _idx_load(j + 2, nb)
                    _gather_start(j + 2, nb)

                @pl.when(j < nch)
                def _():
                    _gather_wait(j, b3)
                    _scatter_start(j, b3)

            return carry

        lax.fori_loop(0, ngrp, group, 0)
        for j in (nch - 3, nch - 2, nch - 1):
            _scatter_wait(j, j % 3)
        plsc.subcore_barrier()

        def wcopy(k, carry):
            ch = k * NS + s

            @pl.when(ch < nrowch)
            def _():
                r0 = ch * 16
                pltpu.sync_copy(acc_sh.at[pl.ds(r0, 16)], zbuf)
                pltpu.sync_copy(zbuf, out_hbm.at[c, pl.ds(r0, 16)])

            return carry

        lax.fori_loop(0, kmax, wcopy, 0)

    return _agg_kernel


CHUNK = 100
NCHK = EPW // CHUNK
_agg128 = _make_agg(D_HID, CHUNK)
_agg48 = _make_agg(D_OUT_PAD, CHUNK)


def _norm(deg):
    return jnp.where(deg > 0, lax.rsqrt(jnp.maximum(deg, 1.0)), 0.0)


_RB = 1000
_GRID = N // _RB


def _tc1_body(x_ref, dp_ref, w1_ref, h1_ref):
    no = _norm(jnp.sum(dp_ref[0], axis=0))
    h1_ref[...] = jnp.dot(
        x_ref[...] * no[:, None], w1_ref[...], preferred_element_type=jnp.float32
    )


def _tc1_call(x, dsrc_p, W1):
    return pl.pallas_call(
        _tc1_body,
        grid=(_GRID,),
        in_specs=[
            pl.BlockSpec((_RB, D_IN), lambda i: (i, 0)),
            pl.BlockSpec((1, NW, _RB), lambda i: (i, 0, 0)),
            pl.BlockSpec((D_IN, D_HID), lambda i: (0, 0)),
        ],
        out_specs=pl.BlockSpec((_RB, D_HID), lambda i: (i, 0)),
        out_shape=jax.ShapeDtypeStruct((N, D_HID), jnp.float32),
    )(x, dsrc_p, W1)


def _tc2_body(agg_ref, dps_ref, dpd_ref, b1_ref, w2_ref, h2_ref):
    agg = agg_ref[0] + agg_ref[1]
    ni = _norm(jnp.sum(dpd_ref[0], axis=0))
    no = _norm(jnp.sum(dps_ref[0], axis=0))
    o1 = jnp.maximum(agg * ni[:, None] + b1_ref[0][None, :], 0.0)
    h2_ref[...] = jnp.dot(
        o1 * no[:, None], w2_ref[...], preferred_element_type=jnp.float32
    )


def _tc2_call(agg1, dsrc_p, ddst_p, b1t, W2p):
    return pl.pallas_call(
        _tc2_body,
        grid=(_GRID,),
        in_specs=[
            pl.BlockSpec((NC, _RB, D_HID), lambda i: (0, i, 0)),
            pl.BlockSpec((1, NW, _RB), lambda i: (i, 0, 0)),
            pl.BlockSpec((1, NW, _RB), lambda i: (i, 0, 0)),
            pl.BlockSpec((8, D_HID), lambda i: (0, 0)),
            pl.BlockSpec((D_HID, D_OUT_PAD), lambda i: (0, 0)),
        ],
        out_specs=pl.BlockSpec((_RB, D_OUT_PAD), lambda i: (i, 0)),
        out_shape=jax.ShapeDtypeStruct((N, D_OUT_PAD), jnp.float32),
    )(agg1, dsrc_p, ddst_p, b1t, W2p)


def _tc3_body(agg_ref, dpd_ref, b2_ref, out_ref):
    agg = agg_ref[0] + agg_ref[1]
    ni = _norm(jnp.sum(dpd_ref[0], axis=0))
    out_ref[...] = agg * ni[:, None] + b2_ref[0][None, :]


def _tc3_call(agg2, ddst_p, b2t):
    return pl.pallas_call(
        _tc3_body,
        grid=(_GRID,),
        in_specs=[
            pl.BlockSpec((NC, _RB, D_OUT_PAD), lambda i: (0, i, 0)),
            pl.BlockSpec((1, NW, _RB), lambda i: (i, 0, 0)),
            pl.BlockSpec((8, D_OUT_PAD), lambda i: (0, 0)),
        ],
        out_specs=pl.BlockSpec((_RB, D_OUT_PAD), lambda i: (i, 0)),
        out_shape=jax.ShapeDtypeStruct((N, D_OUT_PAD), jnp.float32),
    )(agg2, ddst_p, b2t)


@jax.jit
def kernel(x, edge_index, W1, b1, W2, b2):
    src = edge_index[0].astype(jnp.int32)
    dst = edge_index[1].astype(jnp.int32)

    dsrc_p, ddst_p = _degree_kernel(src, dst)
    dsrc_p = dsrc_p.reshape(NW, _GRID, _RB).transpose(1, 0, 2)
    ddst_p = ddst_p.reshape(NW, _GRID, _RB).transpose(1, 0, 2)

    srcr = src.reshape(NW, NCHK, CHUNK)
    dstr = dst.reshape(NW, NCHK, CHUNK)

    h1 = _tc1_call(x, dsrc_p, W1)
    agg1 = _agg128(h1, srcr, dstr)

    W2p = jnp.pad(W2, ((0, 0), (0, D_OUT_PAD - D_OUT)))
    b1t = jnp.tile(b1[None, :], (8, 1))
    h2 = _tc2_call(agg1, dsrc_p, ddst_p, b1t, W2p)
    agg2 = _agg48(h2, srcr, dstr)

    b2t = jnp.tile(jnp.pad(b2, (0, D_OUT_PAD - D_OUT))[None, :], (8, 1))
    outp = _tc3_call(agg2, ddst_p, b2t)
    return outp[:, :D_OUT]


# speedup vs baseline: 14.4605x; 1.4520x over previous
import functools

import jax
import jax.numpy as jnp
from jax import lax
from jax.experimental import pallas as pl
from jax.experimental.pallas import tpu as pltpu
from jax.experimental.pallas import tpu_sc as plsc

N = 10000
E = 320000
D_IN = 128
D_HID = 128
D_OUT = 41
D_OUT_PAD = 48

NC = 2
NS = 16
NW = NC * NS
EPW = E // NW

_mesh = plsc.VectorSubcoreMesh(core_axis_name="c", subcore_axis_name="s")


@functools.partial(
    pl.kernel,
    mesh=_mesh,
    compiler_params=pltpu.CompilerParams(needs_layout_passes=False),
    out_type=[
        jax.ShapeDtypeStruct((NW, N), jnp.float32),
        jax.ShapeDtypeStruct((NW, N), jnp.float32),
    ],
    scratch_types=[
        pltpu.VMEM((EPW,), jnp.int32),
        pltpu.VMEM((EPW,), jnp.int32),
        pltpu.VMEM((N,), jnp.float32),
        pltpu.VMEM((N,), jnp.float32),
    ],
)
def _degree_kernel(src_hbm, dst_hbm, dsrc_out, ddst_out, src_v, dst_v, hs_v, hd_v):
    c = lax.axis_index("c")
    s = lax.axis_index("s")
    wid = s * NC + c
    base = wid * EPW
    pltpu.sync_copy(src_hbm.at[pl.ds(base, EPW)], src_v)
    pltpu.sync_copy(dst_hbm.at[pl.ds(base, EPW)], dst_v)

    zero16 = jnp.zeros((16,), jnp.float32)

    def zbody(i, carry):
        hs_v[pl.ds(i * 16, 16)] = zero16
        hd_v[pl.ds(i * 16, 16)] = zero16
        return carry

    lax.fori_loop(0, N // 16, zbody, 0)

    ones16 = jnp.ones((16,), jnp.float32)

    def body(i, carry):
        sv = src_v[pl.ds(i * 16, 16)]
        dv = dst_v[pl.ds(i * 16, 16)]
        plsc.addupdate_scatter(hs_v, [sv], ones16)
        plsc.addupdate_scatter(hd_v, [dv], ones16)
        return carry

    lax.fori_loop(0, EPW // 16, body, 0)

    pltpu.sync_copy(hs_v, dsrc_out.at[wid])
    pltpu.sync_copy(hd_v, ddst_out.at[wid])


def _make_agg(d, chunk, idxb):
    nch = EPW // chunk
    nbat = nch // idxb
    zr = 80
    nrowch = N // zr
    kmax = -(-nrowch // NS)

    @functools.partial(
        pl.kernel,
        mesh=_mesh,
        compiler_params=pltpu.CompilerParams(
            needs_layout_passes=False, use_tc_tiling_on_sc=False
        ),
        out_type=jax.ShapeDtypeStruct((NC, N, d), jnp.float32),
        scratch_types=[
            pltpu.VMEM((2, idxb, chunk), jnp.int32),
            pltpu.VMEM((2, idxb, chunk), jnp.int32),
            [pltpu.VMEM((chunk, d), jnp.float32) for _ in range(2)],
            pltpu.VMEM((zr, d), jnp.float32),
            pltpu.VMEM_SHARED((N, d), jnp.float32),
            [pltpu.SemaphoreType.DMA for _ in range(2)],
            [pltpu.SemaphoreType.DMA for _ in range(2)],
            [pltpu.SemaphoreType.DMA for _ in range(2)],
        ],
    )
    def _agg_kernel(h_hbm, src_hbm, dst_hbm, out_hbm, src_b, dst_b, rows, zbuf, acc_sh, gsem, ssem, isem):
        c = lax.axis_index("c")
        s = lax.axis_index("s")
        wid = s * NC + c

        def _ibatch_start(k, p):
            pltpu.async_copy(src_hbm.at[wid, pl.ds(k * idxb, idxb)], src_b.at[p], isem[0])
            pltpu.async_copy(dst_hbm.at[wid, pl.ds(k * idxb, idxb)], dst_b.at[p], isem[1])

        def _ibatch_wait(k, p):
            pltpu.make_async_copy(src_hbm.at[wid, pl.ds(k * idxb, idxb)], src_b.at[p], isem[0]).wait()
            pltpu.make_async_copy(dst_hbm.at[wid, pl.ds(k * idxb, idxb)], dst_b.at[p], isem[1]).wait()

        _ibatch_start(0, 0)

        zero16 = jnp.zeros((16,), jnp.float32)
        dl = d // 16

        def zfill(i, carry):
            r = i // dl
            col = lax.rem(i, dl)
            zbuf[r, pl.ds(col * 16, 16)] = zero16
            return carry

        lax.fori_loop(0, zr * dl, zfill, 0)

        def zcopy(k, carry):
            ch = k * NS + s

            @pl.when(ch < nrowch)
            def _():
                pltpu.sync_copy(zbuf, acc_sh.at[pl.ds(ch * zr, zr)])

            return carry

        lax.fori_loop(0, kmax, zcopy, 0)
        plsc.subcore_barrier()

        def _gather_start(p, j2, b):
            pltpu.async_copy(h_hbm.at[src_b.at[p, j2]], rows[b], gsem[b])

        def _gather_wait(p, j2, b):
            pltpu.make_async_copy(h_hbm.at[src_b.at[p, j2]], rows[b], gsem[b]).wait()

        def _scatter_start(p, j2, b):
            pltpu.async_copy(rows[b], acc_sh.at[dst_b.at[p, j2]], ssem[b], add=True)

        def _scatter_wait(p, j2, b):
            pltpu.make_async_copy(rows[b], acc_sh.at[dst_b.at[p, j2]], ssem[b]).wait()

        _ibatch_wait(0, 0)
        _gather_start(0, 0, 0)

        def batch(k, carry):
            p = lax.rem(k, 2)
            first_b = k == 0
            last_b = k == nbat - 1

            for j2 in range(idxb):
                b = j2 % 2
                nb = 1 - b

                if j2 == 0:
                    @pl.when(jnp.logical_not(first_b))
                    def _():
                        _scatter_wait(p, idxb - 1, nb)

                    @pl.when(jnp.logical_not(last_b))
                    def _():
                        _ibatch_start(k + 1, 1 - p)

                    _gather_start(p, j2 + 1, nb)
                elif j2 == idxb - 1:
                    @pl.when(jnp.logical_not(last_b))
                    def _():
                        _scatter_wait(p, j2 - 1, nb)
                        _ibatch_wait(k + 1, 1 - p)
                        _gather_start(1 - p, 0, nb)
                else:
                    _scatter_wait(p, j2 - 1, nb)
                    _gather_start(p, j2 + 1, nb)

                _gather_wait(p, j2, b)
                _scatter_start(p, j2, b)

            return carry

        lax.fori_loop(0, nbat, batch, 0)
        lastp = (nbat - 1) % 2
        _scatter_wait(lastp, idxb - 2, idxb % 2)
        _scatter_wait(lastp, idxb - 1, (idxb - 1) % 2)
        plsc.subcore_barrier()

        def wcopy(k, carry):
            ch = k * NS + s

            @pl.when(ch < nrowch)
            def _():
                r0 = ch * zr
                pltpu.sync_copy(acc_sh.at[pl.ds(r0, zr)], zbuf)
                pltpu.sync_copy(zbuf, out_hbm.at[c, pl.ds(r0, zr)])

            return carry

        lax.fori_loop(0, kmax, wcopy, 0)

    return _agg_kernel


CHUNK = 100
NCHK = EPW // CHUNK
_agg128 = _make_agg(D_HID, CHUNK, 10)
_agg48 = _make_agg(D_OUT_PAD, CHUNK, 10)


def _norm(deg):
    return jnp.where(deg > 0, lax.rsqrt(jnp.maximum(deg, 1.0)), 0.0)


_RB = 1000
_GRID = N // _RB


def _tc1_body(x_ref, dp_ref, w1_ref, h1_ref):
    no = _norm(jnp.sum(dp_ref[0], axis=0))
    h1_ref[...] = jnp.dot(
        x_ref[...] * no[:, None], w1_ref[...], preferred_element_type=jnp.float32
    )


def _tc1_call(x, dsrc_p, W1):
    return pl.pallas_call(
        _tc1_body,
        grid=(_GRID,),
        in_specs=[
            pl.BlockSpec((_RB, D_IN), lambda i: (i, 0)),
            pl.BlockSpec((1, NW, _RB), lambda i: (i, 0, 0)),
            pl.BlockSpec((D_IN, D_HID), lambda i: (0, 0)),
        ],
        out_specs=pl.BlockSpec((_RB, D_HID), lambda i: (i, 0)),
        out_shape=jax.ShapeDtypeStruct((N, D_HID), jnp.float32),
    )(x, dsrc_p, W1)


def _tc2_body(agg_ref, dps_ref, dpd_ref, b1_ref, w2_ref, h2_ref):
    agg = agg_ref[0] + agg_ref[1]
    ni = _norm(jnp.sum(dpd_ref[0], axis=0))
    no = _norm(jnp.sum(dps_ref[0], axis=0))
    o1 = jnp.maximum(agg * ni[:, None] + b1_ref[0][None, :], 0.0)
    h2_ref[...] = jnp.dot(
        o1 * no[:, None], w2_ref[...], preferred_element_type=jnp.float32
    )


def _tc2_call(agg1, dsrc_p, ddst_p, b1t, W2p):
    return pl.pallas_call(
        _tc2_body,
        grid=(_GRID,),
        in_specs=[
            pl.BlockSpec((NC, _RB, D_HID), lambda i: (0, i, 0)),
            pl.BlockSpec((1, NW, _RB), lambda i: (i, 0, 0)),
            pl.BlockSpec((1, NW, _RB), lambda i: (i, 0, 0)),
            pl.BlockSpec((8, D_HID), lambda i: (0, 0)),
            pl.BlockSpec((D_HID, D_OUT_PAD), lambda i: (0, 0)),
        ],
        out_specs=pl.BlockSpec((_RB, D_OUT_PAD), lambda i: (i, 0)),
        out_shape=jax.ShapeDtypeStruct((N, D_OUT_PAD), jnp.float32),
    )(agg1, dsrc_p, ddst_p, b1t, W2p)


def _tc3_body(agg_ref, dpd_ref, b2_ref, out_ref):
    agg = agg_ref[0] + agg_ref[1]
    ni = _norm(jnp.sum(dpd_ref[0], axis=0))
    out_ref[...] = agg * ni[:, None] + b2_ref[0][None, :]


def _tc3_call(agg2, ddst_p, b2t):
    return pl.pallas_call(
        _tc3_body,
        grid=(_GRID,),
        in_specs=[
            pl.BlockSpec((NC, _RB, D_OUT_PAD), lambda i: (0, i, 0)),
            pl.BlockSpec((1, NW, _RB), lambda i: (i, 0, 0)),
            pl.BlockSpec((8, D_OUT_PAD), lambda i: (0, 0)),
        ],
        out_specs=pl.BlockSpec((_RB, D_OUT_PAD), lambda i: (i, 0)),
        out_shape=jax.ShapeDtypeStruct((N, D_OUT_PAD), jnp.float32),
    )(agg2, ddst_p, b2t)


@jax.jit
def kernel(x, edge_index, W1, b1, W2, b2):
    src = edge_index[0].astype(jnp.int32)
    dst = edge_index[1].astype(jnp.int32)

    dsrc_p, ddst_p = _degree_kernel(src, dst)
    dsrc_p = dsrc_p.reshape(NW, _GRID, _RB).transpose(1, 0, 2)
    ddst_p = ddst_p.reshape(NW, _GRID, _RB).transpose(1, 0, 2)

    srcr = src.reshape(NW, NCHK, CHUNK)
    dstr = dst.reshape(NW, NCHK, CHUNK)

    h1 = _tc1_call(x, dsrc_p, W1)
    agg1 = _agg128(h1, srcr, dstr)

    W2p = jnp.pad(W2, ((0, 0), (0, D_OUT_PAD - D_OUT)))
    b1t = jnp.tile(b1[None, :], (8, 1))
    h2 = _tc2_call(agg1, dsrc_p, ddst_p, b1t, W2p)
    agg2 = _agg48(h2, srcr, dstr)

    b2t = jnp.tile(jnp.pad(b2, (0, D_OUT_PAD - D_OUT))[None, :], (8, 1))
    outp = _tc3_call(agg2, ddst_p, b2t)
    return outp[:, :D_OUT]


# speedup vs baseline: 15.8168x; 1.0938x over previous
import functools

import jax
import jax.numpy as jnp
from jax import lax
from jax.experimental import pallas as pl
from jax.experimental.pallas import tpu as pltpu
from jax.experimental.pallas import tpu_sc as plsc

N = 10000
E = 320000
D_IN = 128
D_HID = 128
D_OUT = 41
D_OUT_PAD = 48

NC = 2
NS = 16
NW = NC * NS
EPW = E // NW

_mesh = plsc.VectorSubcoreMesh(core_axis_name="c", subcore_axis_name="s")


@functools.partial(
    pl.kernel,
    mesh=_mesh,
    compiler_params=pltpu.CompilerParams(needs_layout_passes=False),
    out_type=[
        jax.ShapeDtypeStruct((NW, N), jnp.float32),
        jax.ShapeDtypeStruct((NW, N), jnp.float32),
    ],
    scratch_types=[
        pltpu.VMEM((EPW,), jnp.int32),
        pltpu.VMEM((EPW,), jnp.int32),
        pltpu.VMEM((N,), jnp.float32),
        pltpu.VMEM((N,), jnp.float32),
    ],
)
def _degree_kernel(src_hbm, dst_hbm, dsrc_out, ddst_out, src_v, dst_v, hs_v, hd_v):
    c = lax.axis_index("c")
    s = lax.axis_index("s")
    wid = s * NC + c
    base = wid * EPW
    pltpu.sync_copy(src_hbm.at[pl.ds(base, EPW)], src_v)
    pltpu.sync_copy(dst_hbm.at[pl.ds(base, EPW)], dst_v)

    zero16 = jnp.zeros((16,), jnp.float32)

    def zbody(i, carry):
        hs_v[pl.ds(i * 16, 16)] = zero16
        hd_v[pl.ds(i * 16, 16)] = zero16
        return carry

    lax.fori_loop(0, N // 16, zbody, 0)

    ones16 = jnp.ones((16,), jnp.float32)

    def body(i, carry):
        sv = src_v[pl.ds(i * 16, 16)]
        dv = dst_v[pl.ds(i * 16, 16)]
        plsc.addupdate_scatter(hs_v, [sv], ones16)
        plsc.addupdate_scatter(hd_v, [dv], ones16)
        return carry

    lax.fori_loop(0, EPW // 16, body, 0)

    pltpu.sync_copy(hs_v, dsrc_out.at[wid])
    pltpu.sync_copy(hd_v, ddst_out.at[wid])


def _make_agg(d, chunk, idxb, depth):
    nch = EPW // chunk
    nbat = nch // idxb
    zr = 40
    nrowch = N // zr
    kmax = -(-nrowch // NS)
    assert idxb % depth == 0 and nch % idxb == 0 and depth >= 2

    @functools.partial(
        pl.kernel,
        mesh=_mesh,
        compiler_params=pltpu.CompilerParams(
            needs_layout_passes=False, use_tc_tiling_on_sc=False
        ),
        out_type=jax.ShapeDtypeStruct((NC, N, d), jnp.float32),
        scratch_types=[
            pltpu.VMEM((2, idxb, chunk), jnp.int32),
            pltpu.VMEM((2, idxb, chunk), jnp.int32),
            [pltpu.VMEM((chunk, d), jnp.float32) for _ in range(depth)],
            pltpu.VMEM((zr, d), jnp.float32),
            pltpu.VMEM_SHARED((N, d), jnp.float32),
            [pltpu.SemaphoreType.DMA for _ in range(depth)],
            [pltpu.SemaphoreType.DMA for _ in range(depth)],
            [pltpu.SemaphoreType.DMA for _ in range(2)],
        ],
    )
    def _agg_kernel(h_hbm, src_hbm, dst_hbm, out_hbm, src_b, dst_b, rows, zbuf, acc_sh, gsem, ssem, isem):
        c = lax.axis_index("c")
        s = lax.axis_index("s")
        wid = s * NC + c

        def _ibatch_start(k, p):
            pltpu.async_copy(src_hbm.at[wid, pl.ds(k * idxb, idxb)], src_b.at[p], isem[0])
            pltpu.async_copy(dst_hbm.at[wid, pl.ds(k * idxb, idxb)], dst_b.at[p], isem[1])

        def _ibatch_wait(k, p):
            pltpu.make_async_copy(src_hbm.at[wid, pl.ds(k * idxb, idxb)], src_b.at[p], isem[0]).wait()
            pltpu.make_async_copy(dst_hbm.at[wid, pl.ds(k * idxb, idxb)], dst_b.at[p], isem[1]).wait()

        _ibatch_start(0, 0)

        zero16 = jnp.zeros((16,), jnp.float32)
        dl = d // 16

        def zfill(i, carry):
            r = i // dl
            col = lax.rem(i, dl)
            zbuf[r, pl.ds(col * 16, 16)] = zero16
            return carry

        lax.fori_loop(0, zr * dl, zfill, 0)

        def zcopy(k, carry):
            ch = k * NS + s

            @pl.when(ch < nrowch)
            def _():
                pltpu.sync_copy(zbuf, acc_sh.at[pl.ds(ch * zr, zr)])

            return carry

        lax.fori_loop(0, kmax, zcopy, 0)
        plsc.subcore_barrier()

        def _gather_start(p, j2, b):
            pltpu.async_copy(h_hbm.at[src_b.at[p, j2]], rows[b], gsem[b])

        def _gather_wait(p, j2, b):
            pltpu.make_async_copy(h_hbm.at[src_b.at[p, j2]], rows[b], gsem[b]).wait()

        def _scatter_start(p, j2, b):
            pltpu.async_copy(rows[b], acc_sh.at[dst_b.at[p, j2]], ssem[b], add=True)

        def _scatter_wait(p, j2, b):
            pltpu.make_async_copy(rows[b], acc_sh.at[dst_b.at[p, j2]], ssem[b]).wait()

        _ibatch_wait(0, 0)
        for jj in range(depth - 1):
            _gather_start(0, jj, jj)

        def batch(k, carry):
            p = lax.rem(k, 2)
            first_b = k == 0
            last_b = k == nbat - 1

            for j2 in range(idxb):
                b = j2 % depth
                pb = (j2 - 1) % depth

                if j2 == 0:
                    @pl.when(jnp.logical_not(first_b))
                    def _():
                        _scatter_wait(p, idxb - 1, pb)

                    @pl.when(jnp.logical_not(last_b))
                    def _():
                        _ibatch_start(k + 1, 1 - p)
                else:
                    _scatter_wait(p, j2 - 1, pb)

                jg = j2 + depth - 1
                if jg < idxb:
                    _gather_start(p, jg, pb)
                else:
                    if jg == idxb:
                        @pl.when(jnp.logical_not(last_b))
                        def _():
                            _ibatch_wait(k + 1, 1 - p)

                    @pl.when(jnp.logical_not(last_b))
                    def _():
                        _gather_start(1 - p, jg - idxb, pb)

                _gather_wait(p, j2, b)
                _scatter_start(p, j2, b)

            return carry

        lax.fori_loop(0, nbat, batch, 0)
        _scatter_wait((nbat - 1) % 2, idxb - 1, (idxb - 1) % depth)
        plsc.subcore_barrier()

        def wcopy(k, carry):
            ch = k * NS + s

            @pl.when(ch < nrowch)
            def _():
                r0 = ch * zr
                pltpu.sync_copy(acc_sh.at[pl.ds(r0, zr)], zbuf)
                pltpu.sync_copy(zbuf, out_hbm.at[c, pl.ds(r0, zr)])

            return carry

        lax.fori_loop(0, kmax, wcopy, 0)

    return _agg_kernel


CHUNK = 125
NCHK = EPW // CHUNK
_agg128 = _make_agg(D_HID, CHUNK, 8, 2)
_agg48 = _make_agg(D_OUT_PAD, CHUNK, 8, 4)


def _norm(deg):
    return jnp.where(deg > 0, lax.rsqrt(jnp.maximum(deg, 1.0)), 0.0)


_RB = 1000
_GRID = N // _RB


def _tc1_body(x_ref, dp_ref, w1_ref, h1_ref):
    no = _norm(jnp.sum(dp_ref[0], axis=0))
    h1_ref[...] = jnp.dot(
        x_ref[...] * no[:, None], w1_ref[...], preferred_element_type=jnp.float32
    )


def _tc1_call(x, dsrc_p, W1):
    return pl.pallas_call(
        _tc1_body,
        grid=(_GRID,),
        in_specs=[
            pl.BlockSpec((_RB, D_IN), lambda i: (i, 0)),
            pl.BlockSpec((1, NW, _RB), lambda i: (i, 0, 0)),
            pl.BlockSpec((D_IN, D_HID), lambda i: (0, 0)),
        ],
        out_specs=pl.BlockSpec((_RB, D_HID), lambda i: (i, 0)),
        out_shape=jax.ShapeDtypeStruct((N, D_HID), jnp.float32),
    )(x, dsrc_p, W1)


def _tc2_body(agg_ref, dps_ref, dpd_ref, b1_ref, w2_ref, h2_ref):
    agg = agg_ref[0] + agg_ref[1]
    ni = _norm(jnp.sum(dpd_ref[0], axis=0))
    no = _norm(jnp.sum(dps_ref[0], axis=0))
    o1 = jnp.maximum(agg * ni[:, None] + b1_ref[0][None, :], 0.0)
    h2_ref[...] = jnp.dot(
        o1 * no[:, None], w2_ref[...], preferred_element_type=jnp.float32
    )


def _tc2_call(agg1, dsrc_p, ddst_p, b1t, W2p):
    return pl.pallas_call(
        _tc2_body,
        grid=(_GRID,),
        in_specs=[
            pl.BlockSpec((NC, _RB, D_HID), lambda i: (0, i, 0)),
            pl.BlockSpec((1, NW, _RB), lambda i: (i, 0, 0)),
            pl.BlockSpec((1, NW, _RB), lambda i: (i, 0, 0)),
            pl.BlockSpec((8, D_HID), lambda i: (0, 0)),
            pl.BlockSpec((D_HID, D_OUT_PAD), lambda i: (0, 0)),
        ],
        out_specs=pl.BlockSpec((_RB, D_OUT_PAD), lambda i: (i, 0)),
        out_shape=jax.ShapeDtypeStruct((N, D_OUT_PAD), jnp.float32),
    )(agg1, dsrc_p, ddst_p, b1t, W2p)


def _tc3_body(agg_ref, dpd_ref, b2_ref, out_ref):
    agg = agg_ref[0] + agg_ref[1]
    ni = _norm(jnp.sum(dpd_ref[0], axis=0))
    out_ref[...] = agg * ni[:, None] + b2_ref[0][None, :]


def _tc3_call(agg2, ddst_p, b2t):
    return pl.pallas_call(
        _tc3_body,
        grid=(_GRID,),
        in_specs=[
            pl.BlockSpec((NC, _RB, D_OUT_PAD), lambda i: (0, i, 0)),
            pl.BlockSpec((1, NW, _RB), lambda i: (i, 0, 0)),
            pl.BlockSpec((8, D_OUT_PAD), lambda i: (0, 0)),
        ],
        out_specs=pl.BlockSpec((_RB, D_OUT_PAD), lambda i: (i, 0)),
        out_shape=jax.ShapeDtypeStruct((N, D_OUT_PAD), jnp.float32),
    )(agg2, ddst_p, b2t)


@jax.jit
def kernel(x, edge_index, W1, b1, W2, b2):
    src = edge_index[0].astype(jnp.int32)
    dst = edge_index[1].astype(jnp.int32)

    dsrc_p, ddst_p = _degree_kernel(src, dst)
    dsrc_p = dsrc_p.reshape(NW, _GRID, _RB).transpose(1, 0, 2)
    ddst_p = ddst_p.reshape(NW, _GRID, _RB).transpose(1, 0, 2)

    srcr = src.reshape(NW, NCHK, CHUNK)
    dstr = dst.reshape(NW, NCHK, CHUNK)

    h1 = _tc1_call(x, dsrc_p, W1)
    agg1 = _agg128(h1, srcr, dstr)

    W2p = jnp.pad(W2, ((0, 0), (0, D_OUT_PAD - D_OUT)))
    b1t = jnp.tile(b1[None, :], (8, 1))
    h2 = _tc2_call(agg1, dsrc_p, ddst_p, b1t, W2p)
    agg2 = _agg48(h2, srcr, dstr)

    b2t = jnp.tile(jnp.pad(b2, (0, D_OUT_PAD - D_OUT))[None, :], (8, 1))
    outp = _tc3_call(agg2, ddst_p, b2t)
    return outp[:, :D_OUT]
